# baseline (device time: 42440 ns/iter reference)
import jax
import jax.numpy as jnp
from jax import lax
from jax.experimental import pallas as pl
from jax.experimental.pallas import tpu as pltpu

N_DEV = 32
E_LOCAL = 4
N_TOK = 1024
D = 512
H = 1024
ROWS = N_TOK // N_DEV
FOLD = H // 128
BITS = 16
WPR = 128 // BITS
N_WORDS = N_TOK // BITS


def kernel(x, router_W, route_idx, expert_W, shared_W):
    def body(x_ref, rw_ref, idx_ref, idx_smem, r8_ref, ew_ref, sw_ref,
             out_ref, contrib_ref, rrows_ref, pk_vmem, pk_smem,
             send_sems, recv_sems, local_sem):
        my = lax.axis_index("i")

        rrows_ref[...] = jnp.zeros((ROWS, FOLD, 128), jnp.float32)

        r8 = r8_ref[...]
        flat = (128 * lax.broadcasted_iota(jnp.int32, (8, 128), 0)
                + lax.broadcasted_iota(jnp.int32, (8, 128), 1))
        pred8 = jnp.logical_and(r8 // E_LOCAL == my,
                                flat // ROWS != my)
        c = lax.broadcasted_iota(jnp.int32, (128, WPR), 0)
        q = lax.broadcasted_iota(jnp.int32, (128, WPR), 1)
        B = jnp.where(c // BITS == q,
                      jnp.left_shift(jnp.int32(1), c % BITS),
                      0).astype(jnp.float32)
        words = jnp.dot(pred8.astype(jnp.float32), B,
                        preferred_element_type=jnp.float32)
        pk_vmem[...] = words.astype(jnp.int32)
        cp = pltpu.make_async_copy(pk_vmem, pk_smem, local_sem)
        cp.start()
        cp.wait()

        barrier_sem = pltpu.get_barrier_semaphore()
        for o in range(1, N_DEV):
            pl.semaphore_signal(
                barrier_sem, inc=1,
                device_id=((my + o) % N_DEV,),
                device_id_type=pl.DeviceIdType.MESH,
            )
        pl.semaphore_wait(barrier_sem, N_DEV - 1)

        xb = x_ref[...].astype(jnp.bfloat16)
        scores = jnp.dot(xb, rw_ref[...].astype(jnp.bfloat16),
                         preferred_element_type=jnp.float32)
        m = jnp.max(scores, axis=-1, keepdims=True)
        p = jnp.exp(scores - m)
        probs = p / jnp.sum(p, axis=-1, keepdims=True)

        eidx = idx_ref[...]
        col = lax.broadcasted_iota(jnp.int32, (N_TOK, 128), 1)
        p_tok = jnp.sum(jnp.where(col == eidx, probs, 0.0),
                        axis=-1, keepdims=True)

        parts = []
        for k in range(E_LOCAL):
            e = my * E_LOCAL + k
            w_k = jnp.where(eidx == e, p_tok, 0.0)
            parts.append(xb * w_k.astype(jnp.bfloat16))
        xw_all = jnp.concatenate(parts, axis=1)
        w_all = ew_ref[...].astype(jnp.bfloat16).reshape(E_LOCAL * D, H)
        acc = jnp.dot(xw_all, w_all,
                      preferred_element_type=jnp.float32)
        for s in range(FOLD):
            contrib_ref[:, s, :] = acc[:, s * 128:(s + 1) * 128]

        for w in range(N_WORDS):
            word = pk_smem[w // WPR, w % WPR]

            @pl.when(word != 0)
            def _(w=w, word=word):
                for b in range(BITS):
                    i = 128 * (w // WPR) + BITS * (w % WPR) + b
                    j = i // ROWS
                    r = i - j * ROWS

                    @pl.when(lax.shift_right_logical(word, b) & 1 == 1)
                    def _(i=i, j=j, r=r):
                        pltpu.make_async_remote_copy(
                            src_ref=contrib_ref.at[i],
                            dst_ref=rrows_ref.at[r],
                            send_sem=send_sems.at[0],
                            recv_sem=recv_sems.at[r],
                            device_id=(j,),
                            device_id_type=pl.DeviceIdType.MESH,
                        ).start()

        x_own = x_ref[pl.ds(my * ROWS, ROWS), :].astype(jnp.bfloat16)
        shared_own = jnp.dot(x_own, sw_ref[...].astype(jnp.bfloat16),
                             preferred_element_type=jnp.float32)

        for r in range(ROWS):
            s_dev = idx_smem[my * ROWS + r, 0] // E_LOCAL

            @pl.when(s_dev != my)
            def _(r=r, s_dev=s_dev):
                pltpu.make_async_remote_copy(
                    src_ref=contrib_ref.at[0],
                    dst_ref=rrows_ref.at[r],
                    send_sem=send_sems.at[0],
                    recv_sem=recv_sems.at[r],
                    device_id=(s_dev,),
                    device_id_type=pl.DeviceIdType.MESH,
                ).wait_recv()

        recv_parts = [rrows_ref[:, s, :] for s in range(FOLD)]
        recv_rows = jnp.concatenate(recv_parts, axis=-1)
        own_parts = [contrib_ref[pl.ds(my * ROWS, ROWS), s, :]
                     for s in range(FOLD)]
        own_rows = jnp.concatenate(own_parts, axis=-1)
        out_ref[...] = shared_own + own_rows + recv_rows

        for w in range(N_WORDS):
            word = pk_smem[w // WPR, w % WPR]

            @pl.when(word != 0)
            def _(w=w, word=word):
                for b in range(BITS):
                    @pl.when(lax.shift_right_logical(word, b) & 1 == 1)
                    def _():
                        pltpu.make_async_remote_copy(
                            src_ref=contrib_ref.at[0],
                            dst_ref=rrows_ref.at[0],
                            send_sem=send_sems.at[0],
                            recv_sem=recv_sems.at[0],
                            device_id=(my,),
                            device_id_type=pl.DeviceIdType.MESH,
                        ).wait_send()

    return pl.pallas_call(
        body,
        out_shape=jax.ShapeDtypeStruct((ROWS, H), jnp.float32),
        in_specs=[
            pl.BlockSpec(memory_space=pltpu.VMEM),
            pl.BlockSpec(memory_space=pltpu.VMEM),
            pl.BlockSpec(memory_space=pltpu.VMEM),
            pl.BlockSpec(memory_space=pltpu.SMEM),
            pl.BlockSpec(memory_space=pltpu.VMEM),
            pl.BlockSpec(memory_space=pltpu.VMEM),
            pl.BlockSpec(memory_space=pltpu.VMEM),
        ],
        out_specs=pl.BlockSpec(memory_space=pltpu.VMEM),
        scratch_shapes=[
            pltpu.VMEM((N_TOK, FOLD, 128), jnp.float32),
            pltpu.VMEM((ROWS, FOLD, 128), jnp.float32),
            pltpu.VMEM((8, WPR), jnp.int32),
            pltpu.SMEM((8, WPR), jnp.int32),
            pltpu.SemaphoreType.DMA((1,)),
            pltpu.SemaphoreType.DMA((ROWS,)),
            pltpu.SemaphoreType.DMA,
        ],
        compiler_params=pltpu.CompilerParams(collective_id=0),
    )(x, router_W, route_idx, route_idx, route_idx.reshape(8, 128),
      expert_W, shared_W)


# device time: 30487 ns/iter; 1.3921x vs baseline; 1.3921x over previous
import jax
import jax.numpy as jnp
from jax import lax
from jax.experimental import pallas as pl
from jax.experimental.pallas import tpu as pltpu

N_DEV = 32
E_LOCAL = 4
N_TOK = 1024
D = 512
H = 1024
ROWS = N_TOK // N_DEV
FOLD = H // 128
BITS = 16
WPR = 128 // BITS
N_WORDS = N_TOK // BITS
GRP = 8


def kernel(x, router_W, route_idx, expert_W, shared_W):
    def body(x_ref, rw_ref, idx_ref, r8_ref, ew_ref, sw_ref,
             out_ref, contrib_ref, rrows_ref, pk_vmem, pk_smem,
             send_sems, recv_sems, local_sem):
        my = lax.axis_index("i")

        barrier_sem = pltpu.get_barrier_semaphore()
        grp = my // GRP
        pos = my % GRP
        for o in range(1, GRP):
            pl.semaphore_signal(
                barrier_sem, inc=GRP,
                device_id=(grp * GRP + (pos + o) % GRP,),
                device_id_type=pl.DeviceIdType.MESH,
            )

        r8 = r8_ref[...]
        flat = (128 * lax.broadcasted_iota(jnp.int32, (8, 128), 0)
                + lax.broadcasted_iota(jnp.int32, (8, 128), 1))
        contrib_dev = r8 // E_LOCAL
        owner_dev = flat // ROWS
        pred_send = jnp.logical_and(contrib_dev == my, owner_dev != my)
        pred_recv = contrib_dev != owner_dev
        c = lax.broadcasted_iota(jnp.int32, (128, WPR), 0)
        q = lax.broadcasted_iota(jnp.int32, (128, WPR), 1)
        B = jnp.where(c // BITS == q,
                      jnp.left_shift(jnp.int32(1), c % BITS),
                      0).astype(jnp.float32)
        ws = jnp.dot(pred_send.astype(jnp.float32), B,
                     preferred_element_type=jnp.float32)
        wr = jnp.dot(pred_recv.astype(jnp.float32), B,
                     preferred_element_type=jnp.float32)
        pk_vmem[...] = jnp.concatenate([ws, wr], axis=1).astype(jnp.int32)
        cp = pltpu.make_async_copy(pk_vmem, pk_smem, local_sem)
        cp.start()

        xb = x_ref[...].astype(jnp.bfloat16)
        scores = jnp.dot(xb, rw_ref[...].astype(jnp.bfloat16),
                         preferred_element_type=jnp.float32)
        m = jnp.max(scores, axis=-1, keepdims=True)
        p = jnp.exp(scores - m)
        probs = p / jnp.sum(p, axis=-1, keepdims=True)

        eidx = idx_ref[...]
        col = lax.broadcasted_iota(jnp.int32, (N_TOK, 128), 1)
        p_tok = jnp.sum(jnp.where(col == eidx, probs, 0.0),
                        axis=-1, keepdims=True)

        acc = jnp.zeros((N_TOK, H), jnp.float32)
        for k in range(E_LOCAL):
            e = my * E_LOCAL + k
            w_k = jnp.where(eidx == e, p_tok, 0.0)
            xw = xb * w_k.astype(jnp.bfloat16)
            acc = acc + jnp.dot(xw, ew_ref[k].astype(jnp.bfloat16),
                                preferred_element_type=jnp.float32)
        contrib_ref[...] = acc.reshape(N_TOK, FOLD, 128)

        pl.semaphore_wait(barrier_sem, (GRP - 1) * GRP)
        for g in range(1, N_DEV // GRP):
            pl.semaphore_signal(
                barrier_sem, inc=1,
                device_id=(((grp + g) % (N_DEV // GRP)) * GRP + pos,),
                device_id_type=pl.DeviceIdType.MESH,
            )
        pl.semaphore_wait(barrier_sem, N_DEV // GRP - 1)


        cp.wait()

        for w in range(N_WORDS):
            word = pk_smem[w // WPR, w % WPR]

            @pl.when(word != 0)
            def _(w=w, word=word):
                for b in range(BITS):
                    i = 128 * (w // WPR) + BITS * (w % WPR) + b
                    j = i // ROWS
                    r = i - j * ROWS

                    @pl.when(lax.shift_right_logical(word, b) & 1 == 1)
                    def _(i=i, j=j, r=r):
                        pltpu.make_async_remote_copy(
                            src_ref=contrib_ref.at[i],
                            dst_ref=rrows_ref.at[r],
                            send_sem=send_sems.at[0],
                            recv_sem=recv_sems.at[r],
                            device_id=(j,),
                            device_id_type=pl.DeviceIdType.MESH,
                        ).start()

        x_own = x_ref[pl.ds(my * ROWS, ROWS), :].astype(jnp.bfloat16)
        shared_own = jnp.dot(x_own, sw_ref[...].astype(jnp.bfloat16),
                             preferred_element_type=jnp.float32)

        for u in range(2):
            wu = 2 * my + u
            word = pk_smem[wu // WPR, WPR + wu % WPR]
            for b in range(BITS):
                r = BITS * u + b

                @pl.when(lax.shift_right_logical(word, b) & 1 == 1)
                def _(r=r):
                    pltpu.make_async_remote_copy(
                        src_ref=contrib_ref.at[0],
                        dst_ref=rrows_ref.at[r],
                        send_sem=send_sems.at[0],
                        recv_sem=recv_sems.at[r],
                        device_id=(0,),
                        device_id_type=pl.DeviceIdType.MESH,
                    ).wait_recv()

        cont_my = idx_ref[pl.ds(my * ROWS, ROWS), :] // E_LOCAL
        recv_rows = jnp.where(cont_my != my,
                              rrows_ref[...].reshape(ROWS, H),
                              0.0)
        own_rows = contrib_ref[pl.ds(my * ROWS, ROWS)].reshape(ROWS, H)
        out_ref[...] = shared_own + own_rows + recv_rows

        for w in range(N_WORDS):
            word = pk_smem[w // WPR, w % WPR]

            @pl.when(word != 0)
            def _(w=w, word=word):
                for b in range(BITS):
                    @pl.when(lax.shift_right_logical(word, b) & 1 == 1)
                    def _():
                        pltpu.make_async_remote_copy(
                            src_ref=contrib_ref.at[0],
                            dst_ref=rrows_ref.at[0],
                            send_sem=send_sems.at[0],
                            recv_sem=recv_sems.at[0],
                            device_id=(my,),
                            device_id_type=pl.DeviceIdType.MESH,
                        ).wait_send()

    return pl.pallas_call(
        body,
        out_shape=jax.ShapeDtypeStruct((ROWS, H), jnp.float32),
        in_specs=[
            pl.BlockSpec(memory_space=pltpu.VMEM),
            pl.BlockSpec(memory_space=pltpu.VMEM),
            pl.BlockSpec(memory_space=pltpu.VMEM),
            pl.BlockSpec(memory_space=pltpu.VMEM),
            pl.BlockSpec(memory_space=pltpu.VMEM),
            pl.BlockSpec(memory_space=pltpu.VMEM),
        ],
        out_specs=pl.BlockSpec(memory_space=pltpu.VMEM),
        scratch_shapes=[
            pltpu.VMEM((N_TOK, FOLD, 128), jnp.float32),
            pltpu.VMEM((ROWS, FOLD, 128), jnp.float32),
            pltpu.VMEM((8, 2 * WPR), jnp.int32),
            pltpu.SMEM((8, 2 * WPR), jnp.int32),
            pltpu.SemaphoreType.DMA((1,)),
            pltpu.SemaphoreType.DMA((ROWS,)),
            pltpu.SemaphoreType.DMA,
        ],
        compiler_params=pltpu.CompilerParams(collective_id=0),
    )(x, router_W, route_idx, route_idx.reshape(8, 128), expert_W, shared_W)
